# Initial kernel scaffold; baseline (speedup 1.0000x reference)
import jax
import jax.numpy as jnp
from jax.experimental import pallas as pl


def kernel(h, edge_index, norm, W, b):
    raise NotImplementedError("write your pallas kernel here")


# speedup vs baseline: 24.8593x; 24.8593x over previous
import functools

import jax
import jax.numpy as jnp
from jax import lax
from jax.experimental import pallas as pl
from jax.experimental.pallas import tpu as pltpu
from jax.experimental.pallas import tpu_sc as plsc

N = 10000
D = 128
E = 320000
CHUNK = 125
NCHUNKS = E // CHUNK
NC, NS = 2, 16
NW = NC * NS
CPT = NCHUNKS // NW
PASSES = 2
CPP = CPT // PASSES
NPAD = 10240
RPS = NPAD // NS
ZROWS = 16
LANES = 16


def _matmul_body(h_ref, w_ref, norm_ref, x_ref):
    x_ref[...] = (
        jnp.dot(h_ref[...], w_ref[...], preferred_element_type=jnp.float32)
        * norm_ref[...]
    )


def _xw_norm(h, W, norm):
    mb = 1000
    return pl.pallas_call(
        _matmul_body,
        grid=(N // mb,),
        in_specs=[
            pl.BlockSpec((mb, D), lambda i: (i, 0)),
            pl.BlockSpec((D, D), lambda i: (0, 0)),
            pl.BlockSpec((mb, 1), lambda i: (i, 0)),
        ],
        out_specs=pl.BlockSpec((mb, D), lambda i: (i, 0)),
        out_shape=jax.ShapeDtypeStruct((N, D), jnp.float32),
    )(h, W, norm)


def _sc_body(x_hbm, src_hbm, dst_hbm, out_hbm,
             sidx, didx, rows0, rows1, zbuf, acc, sem0, sem1):
    c = lax.axis_index("c")
    s = lax.axis_index("s")
    wid = c * NS + s

    zeros = jnp.zeros((LANES,), jnp.float32)

    def zrow(r, carry):
        for k in range(D // LANES):
            zbuf[r, pl.ds(k * LANES, LANES)] = zeros
        return carry

    lax.fori_loop(0, ZROWS, zrow, 0)

    def zcopy(i, carry):
        pltpu.sync_copy(zbuf, acc.at[pl.ds(s * RPS + i * ZROWS, ZROWS)])
        return carry

    lax.fori_loop(0, RPS // ZROWS, zcopy, 0)
    plsc.subcore_barrier()

    rows = (rows0, rows1)
    sems = (sem0, sem1)

    for p in range(PASSES):
        base = wid * CPT + p * CPP
        pltpu.sync_copy(src_hbm.at[pl.ds(base, CPP)], sidx)
        pltpu.sync_copy(dst_hbm.at[pl.ds(base, CPP)], didx)

        pltpu.async_copy(x_hbm.at[sidx.at[0]], rows0, sem0)
        pltpu.async_copy(x_hbm.at[sidx.at[1]], rows1, sem1)

        def body(g, carry):
            for b in range(2):
                j = g * 2 + b
                pltpu.make_async_copy(
                    x_hbm.at[sidx.at[j]], rows[b], sems[b]).wait()
                pltpu.sync_copy(rows[b], acc.at[didx.at[j]], add=True)

                @pl.when(j + 2 < CPP)
                def _():
                    pltpu.async_copy(x_hbm.at[sidx.at[j + 2]], rows[b], sems[b])

            return carry

        lax.fori_loop(0, CPP // 2, body, 0)

    plsc.subcore_barrier()
    pltpu.sync_copy(
        acc.at[pl.ds(s * RPS, RPS)],
        out_hbm.at[c].at[pl.ds(s * RPS, RPS)],
    )


def _sc_segment_sum(x, src_chunks, dst_chunks):
    mesh = plsc.VectorSubcoreMesh(core_axis_name="c", subcore_axis_name="s")
    f = functools.partial(
        pl.kernel,
        out_type=jax.ShapeDtypeStruct((NC, NPAD, D), jnp.float32),
        mesh=mesh,
        scratch_types=[
            pltpu.VMEM((CPP, CHUNK), jnp.int32),
            pltpu.VMEM((CPP, CHUNK), jnp.int32),
            pltpu.VMEM((CHUNK, D), jnp.float32),
            pltpu.VMEM((CHUNK, D), jnp.float32),
            pltpu.VMEM((ZROWS, D), jnp.float32),
            pltpu.VMEM_SHARED((NPAD, D), jnp.float32),
            pltpu.SemaphoreType.DMA,
            pltpu.SemaphoreType.DMA,
        ],
    )(_sc_body)
    return f(x, src_chunks, dst_chunks)


def _combine_body(p_ref, norm_ref, b_ref, o_ref):
    o_ref[...] = jnp.maximum(
        (p_ref[0] + p_ref[1]) * norm_ref[...] + b_ref[...], 0.0
    )


def _combine(partials, norm, b2d):
    mb = 1000
    return pl.pallas_call(
        _combine_body,
        grid=(N // mb,),
        in_specs=[
            pl.BlockSpec((NC, mb, D), lambda i: (0, i, 0)),
            pl.BlockSpec((mb, 1), lambda i: (i, 0)),
            pl.BlockSpec((1, D), lambda i: (0, 0)),
        ],
        out_specs=pl.BlockSpec((mb, D), lambda i: (i, 0)),
        out_shape=jax.ShapeDtypeStruct((N, D), jnp.float32),
    )(partials, norm, b2d)


def kernel(h, edge_index, norm, W, b):
    x = _xw_norm(h, W, norm)
    src_chunks = edge_index[0].reshape(NCHUNKS, CHUNK)
    dst_chunks = edge_index[1].reshape(NCHUNKS, CHUNK)
    partials = _sc_segment_sum(x, src_chunks, dst_chunks)
    return _combine(partials, norm, b.reshape(1, D))
